# baseline (speedup 1.0000x reference)
import jax
import jax.numpy as jnp
from jax.experimental import pallas as pl

D_MODEL = 2048
NUM_EXPERTS = 64
NUM_TOKENS = 16384
TW = 64
BM = 512


def _router_tile(x_ref, wt_ref, b_ref, t_ref, w_ref):
    logits = jax.lax.dot_general(
        x_ref[...].astype(jnp.bfloat16), wt_ref[...].astype(jnp.bfloat16),
        dimension_numbers=(((1,), (0,)), ((), ())),
        preferred_element_type=jnp.float32,
    ) + b_ref[...]
    m = jnp.max(logits, axis=-1, keepdims=True)
    e = jnp.exp(logits - m)
    s = jnp.sum(e, axis=-1, keepdims=True)
    w = e / s
    w_ref[...] = w
    t = jnp.argmax(w, axis=-1).astype(jnp.int32)
    t_ref[...] = t.reshape(BM // TW, TW)


def kernel(x, W, b):
    wt = W.T
    b2 = b.reshape(1, NUM_EXPERTS)
    grid = (NUM_TOKENS // BM,)
    top1, weights = pl.pallas_call(
        _router_tile,
        grid=grid,
        in_specs=[
            pl.BlockSpec((BM, D_MODEL), lambda i: (i, 0)),
            pl.BlockSpec((D_MODEL, NUM_EXPERTS), lambda i: (0, 0)),
            pl.BlockSpec((1, NUM_EXPERTS), lambda i: (0, 0)),
        ],
        out_specs=[
            pl.BlockSpec((BM // TW, TW), lambda i: (i, 0)),
            pl.BlockSpec((BM, NUM_EXPERTS), lambda i: (i, 0)),
        ],
        out_shape=[
            jax.ShapeDtypeStruct((NUM_TOKENS // TW, TW), jnp.int32),
            jax.ShapeDtypeStruct((NUM_TOKENS, NUM_EXPERTS), jnp.float32),
        ],
    )(x, wt, b2)
    return top1.reshape(NUM_TOKENS), weights


# speedup vs baseline: 1.1991x; 1.1991x over previous
import jax
import jax.numpy as jnp
from jax.experimental import pallas as pl

D_MODEL = 2048
NUM_EXPERTS = 64
NUM_TOKENS = 16384
LANE = 128
BM = 2048


def _router_tile(x_ref, wt_ref, b_ref, t_ref, w_ref):
    logits = jax.lax.dot_general(
        x_ref[...].astype(jnp.bfloat16), wt_ref[...].astype(jnp.bfloat16),
        dimension_numbers=(((1,), (0,)), ((), ())),
        preferred_element_type=jnp.float32,
    ) + b_ref[...]
    m = jnp.max(logits, axis=-1, keepdims=True)
    e = jnp.exp(logits - m)
    s = jnp.sum(e, axis=-1, keepdims=True)
    w = e / s
    w_ref[...] = w
    t = jnp.argmax(w, axis=-1).astype(jnp.int32)
    t_ref[...] = t.reshape(BM // LANE, LANE)


def kernel(x, W, b):
    wt = W.T
    b2 = b.reshape(1, NUM_EXPERTS)
    grid = (NUM_TOKENS // BM,)
    top1, weights = pl.pallas_call(
        _router_tile,
        grid=grid,
        in_specs=[
            pl.BlockSpec((BM, D_MODEL), lambda i: (i, 0)),
            pl.BlockSpec((D_MODEL, NUM_EXPERTS), lambda i: (0, 0)),
            pl.BlockSpec((1, NUM_EXPERTS), lambda i: (0, 0)),
        ],
        out_specs=[
            pl.BlockSpec((BM // LANE, LANE), lambda i: (i, 0)),
            pl.BlockSpec((BM, NUM_EXPERTS), lambda i: (i, 0)),
        ],
        out_shape=[
            jax.ShapeDtypeStruct((NUM_TOKENS // LANE, LANE), jnp.int32),
            jax.ShapeDtypeStruct((NUM_TOKENS, NUM_EXPERTS), jnp.float32),
        ],
    )(x, wt, b2)
    return top1.reshape(NUM_TOKENS), weights


# speedup vs baseline: 1.2038x; 1.0039x over previous
import jax
import jax.numpy as jnp
from jax.experimental import pallas as pl

D_MODEL = 2048
NUM_EXPERTS = 64
NUM_TOKENS = 16384
LANE = 128
BM = 1024


def _router_tile(x_ref, wt_ref, b_ref, t_ref, w_ref):
    logits = jax.lax.dot_general(
        x_ref[...].astype(jnp.bfloat16), wt_ref[...].astype(jnp.bfloat16),
        dimension_numbers=(((1,), (0,)), ((), ())),
        preferred_element_type=jnp.float32,
    ) + b_ref[...]
    m = jnp.max(logits, axis=-1, keepdims=True)
    e = jnp.exp(logits - m)
    s = jnp.sum(e, axis=-1, keepdims=True)
    w = e / s
    w_ref[...] = w
    t = jnp.argmax(w, axis=-1).astype(jnp.int32)
    t_ref[...] = t.reshape(BM // LANE, LANE)


def kernel(x, W, b):
    wt = W.T
    b2 = b.reshape(1, NUM_EXPERTS)
    grid = (NUM_TOKENS // BM,)
    top1, weights = pl.pallas_call(
        _router_tile,
        grid=grid,
        in_specs=[
            pl.BlockSpec((BM, D_MODEL), lambda i: (i, 0)),
            pl.BlockSpec((D_MODEL, NUM_EXPERTS), lambda i: (0, 0)),
            pl.BlockSpec((1, NUM_EXPERTS), lambda i: (0, 0)),
        ],
        out_specs=[
            pl.BlockSpec((BM // LANE, LANE), lambda i: (i, 0)),
            pl.BlockSpec((BM, NUM_EXPERTS), lambda i: (i, 0)),
        ],
        out_shape=[
            jax.ShapeDtypeStruct((NUM_TOKENS // LANE, LANE), jnp.int32),
            jax.ShapeDtypeStruct((NUM_TOKENS, NUM_EXPERTS), jnp.float32),
        ],
    )(x, wt, b2)
    return top1.reshape(NUM_TOKENS), weights


# speedup vs baseline: 1.2612x; 1.0478x over previous
import jax
import jax.numpy as jnp
from jax.experimental import pallas as pl

D_MODEL = 2048
NUM_EXPERTS = 64
NUM_TOKENS = 16384
BM = 1024


def _router_tile(x_ref, w_mat_ref, b_ref, t_ref, w_ref):
    logits = jax.lax.dot_general(
        x_ref[...].astype(jnp.bfloat16), w_mat_ref[...].astype(jnp.bfloat16),
        dimension_numbers=(((1,), (1,)), ((), ())),
        preferred_element_type=jnp.float32,
    ) + b_ref[...]
    m = jnp.max(logits, axis=-1, keepdims=True)
    e = jnp.exp(logits - m)
    s = jnp.sum(e, axis=-1, keepdims=True)
    w = e / s
    w_ref[...] = w
    t = jnp.argmax(w, axis=-1).astype(jnp.int32)
    t_ref[...] = t.reshape(BM // 128, 128)


def kernel(x, W, b):
    b2 = b.reshape(1, NUM_EXPERTS)
    grid = (NUM_TOKENS // BM,)
    top1, weights = pl.pallas_call(
        _router_tile,
        grid=grid,
        in_specs=[
            pl.BlockSpec((BM, D_MODEL), lambda i: (i, 0)),
            pl.BlockSpec((NUM_EXPERTS, D_MODEL), lambda i: (0, 0)),
            pl.BlockSpec((1, NUM_EXPERTS), lambda i: (0, 0)),
        ],
        out_specs=[
            pl.BlockSpec((BM // 128, 128), lambda i: (i, 0)),
            pl.BlockSpec((BM, NUM_EXPERTS), lambda i: (i, 0)),
        ],
        out_shape=[
            jax.ShapeDtypeStruct((NUM_TOKENS // 128, 128), jnp.int32),
            jax.ShapeDtypeStruct((NUM_TOKENS, NUM_EXPERTS), jnp.float32),
        ],
    )(x, W, b2)
    return top1.reshape(NUM_TOKENS), weights
